# baseline (speedup 1.0000x reference)
import functools

import jax
import jax.numpy as jnp
from jax import lax
from jax.experimental import pallas as pl
from jax.experimental.pallas import tpu as pltpu
from jax.experimental.pallas import tpu_sc as plsc

_CHUNK = 128
_NBUF = 4
_PADW = 128
_LANE = 128


def _pad8(n):
    return (n + 7) // 8 * 8


@functools.lru_cache(maxsize=None)
def _build_transpose(hidden, vocab):
    info = plsc.get_sparse_core_info()
    nc, ns = info.num_cores, info.num_subcores
    nw = nc * ns
    nslab = vocab // _LANE
    tail = vocab - nslab * _LANE
    prow = _LANE // 2

    mesh = plsc.VectorSubcoreMesh(core_axis_name="c", subcore_axis_name="s")

    @functools.partial(
        pl.kernel,
        mesh=mesh,
        out_type=jax.ShapeDtypeStruct((vocab // 2, 2 * hidden), jnp.float32),
        compiler_params=pltpu.CompilerParams(use_tc_tiling_on_sc=True, needs_layout_passes=False),
        scratch_types=[
            pltpu.VMEM((hidden, _LANE), jnp.float32),
            pltpu.VMEM((hidden, _LANE), jnp.float32),
            pltpu.VMEM((tail // 2 if tail else 2, _LANE), jnp.float32),
            pltpu.SemaphoreType.DMA,
            pltpu.SemaphoreType.DMA,
        ],
    )
    def transpose_k(tt_hbm, tail_hbm, out_hbm, src_v, comp_v, tail_v, isem, osem):
        wid = lax.axis_index("s") * nc + lax.axis_index("c")

        iota = lax.iota(jnp.int32, 16)
        rvecs = [iota + 16 * t for t in range(8)]
        r2s = [v >> 1 for v in rvecs]
        rp64s = [(v & 1) * hidden for v in rvecs]

        nmine = (nslab - wid + nw - 1) // nw

        def do_slab(k, carry):
            s = wid + k * nw
            pltpu.async_copy(tt_hbm.at[:, pl.ds(s * _LANE, _LANE)], src_v, isem).wait()

            def do_col(c, carry2):
                csplat = jnp.zeros((16,), jnp.int32) + c
                for t in range(8):
                    v = plsc.load_gather(src_v, [csplat, rvecs[t]])
                    plsc.store_scatter(comp_v, [r2s[t], rp64s[t] + c], v)
                return carry2

            lax.fori_loop(0, hidden, do_col, 0)
            pltpu.async_copy(comp_v, out_hbm.at[pl.ds(s * prow, prow)], osem).wait()
            return carry

        lax.fori_loop(0, nmine, do_slab, 0)

        if tail:
            @pl.when(wid == 0)
            def _():
                pltpu.sync_copy(tail_hbm, tail_v)
                pltpu.sync_copy(tail_v, out_hbm.at[pl.ds(nslab * 64, tail // 2)])

    return transpose_k


@functools.lru_cache(maxsize=None)
def _build_gather(total, hidden, vocab):
    info = plsc.get_sparse_core_info()
    nc, ns = info.num_cores, info.num_subcores
    nw = nc * ns
    per_w = total // nw
    nchunk = per_w // _CHUNK
    assert per_w * nw == total and nchunk * _CHUNK == per_w

    mesh = plsc.VectorSubcoreMesh(core_axis_name="c", subcore_axis_name="s")

    @functools.partial(
        pl.kernel,
        mesh=mesh,
        out_type=jax.ShapeDtypeStruct((total // _CHUNK, _CHUNK, _PADW), jnp.float32),
        compiler_params=pltpu.CompilerParams(use_tc_tiling_on_sc=True),
        scratch_types=[
            pltpu.VMEM((_pad8(nchunk), _CHUNK), jnp.int32),
            pltpu.VMEM((_NBUF, _CHUNK, _PADW), jnp.float32),
            pltpu.SemaphoreType.DMA((_NBUF,)),
        ],
    )
    def gather_k(table_hbm, idx_hbm, out_hbm, idx_v, rows_v, gsem):
        wid = lax.axis_index("s") * nc + lax.axis_index("c")
        cbase = wid * nchunk
        pltpu.sync_copy(idx_hbm.at[wid], idx_v)

        for j in range(_NBUF):
            pltpu.async_copy(table_hbm.at[idx_v.at[j]], rows_v.at[j], gsem.at[j])

        def body(c, carry):
            p = lax.rem(c, _NBUF)
            pltpu.make_async_copy(table_hbm.at[idx_v.at[p]], rows_v.at[p], gsem.at[p]).wait()
            pltpu.sync_copy(rows_v.at[p], out_hbm.at[cbase + c])
            pltpu.async_copy(table_hbm.at[idx_v.at[c + _NBUF]], rows_v.at[p], gsem.at[p])
            return carry

        lax.fori_loop(0, nchunk - _NBUF, body, 0)

        for j in range(nchunk - _NBUF, nchunk):
            p = j % _NBUF
            pltpu.make_async_copy(table_hbm.at[idx_v.at[p]], rows_v.at[p], gsem.at[p]).wait()
            pltpu.sync_copy(rows_v.at[p], out_hbm.at[cbase + j])

    return gather_k


def kernel(inputs, embedding):
    b, h = inputs.shape
    total = b * h
    vocab, hidden = embedding.shape
    nw = 32
    nchunk = total // nw // _CHUNK
    nslab = vocab // _LANE
    tail = vocab - nslab * _LANE

    tt = embedding.T
    tail_op = embedding[nslab * _LANE:].reshape(tail // 2 if tail else 2, _LANE)
    table2 = _build_transpose(hidden, vocab)(tt, tail_op)

    idx = inputs.astype(jnp.int32).reshape(total)
    half3 = (idx >> 1).reshape(nw, nchunk, _CHUNK)
    half3 = jnp.pad(half3, ((0, 0), (0, _pad8(nchunk) - nchunk), (0, 0)))
    pairs = _build_gather(total, hidden, vocab)(table2, half3)
    parity = (idx & 1).reshape(total // _CHUNK, _CHUNK, 1)
    out = jnp.where(parity == 1, pairs[:, :, hidden:], pairs[:, :, :hidden])
    return out.reshape(b, h, hidden)


# speedup vs baseline: 1.2287x; 1.2287x over previous
import functools

import jax
import jax.numpy as jnp
from jax import lax
from jax.experimental import pallas as pl
from jax.experimental.pallas import tpu as pltpu
from jax.experimental.pallas import tpu_sc as plsc

_CHUNK = 128
_NBUF = 4
_PADW = 128
_LANE = 128


def _pad8(n):
    return (n + 7) // 8 * 8


@functools.lru_cache(maxsize=None)
def _build_transpose(hidden, vocab):
    info = plsc.get_sparse_core_info()
    nc, ns = info.num_cores, info.num_subcores
    nw = nc * ns
    nslab = vocab // _LANE
    tail = vocab - nslab * _LANE
    prow = _LANE // 2

    mesh = plsc.VectorSubcoreMesh(core_axis_name="c", subcore_axis_name="s")

    @functools.partial(
        pl.kernel,
        mesh=mesh,
        out_type=jax.ShapeDtypeStruct((vocab // 2, 2 * hidden), jnp.float32),
        compiler_params=pltpu.CompilerParams(use_tc_tiling_on_sc=True, needs_layout_passes=False),
        scratch_types=[
            pltpu.VMEM((2, hidden, _LANE), jnp.float32),
            pltpu.VMEM((2, hidden, _LANE), jnp.float32),
            pltpu.VMEM((tail // 2 if tail else 2, _LANE), jnp.float32),
            pltpu.SemaphoreType.DMA((2,)),
            pltpu.SemaphoreType.DMA((2,)),
        ],
    )
    def transpose_k(tt_hbm, tail_hbm, out_hbm, src_v, comp_v, tail_v, isem, osem):
        wid = lax.axis_index("s") * nc + lax.axis_index("c")

        iota = lax.iota(jnp.int32, 16)
        rvecs = [iota + 16 * t for t in range(8)]
        r2s = [v >> 1 for v in rvecs]
        rp64s = [(v & 1) * hidden for v in rvecs]

        nmine = (nslab - wid + nw - 1) // nw

        for j in range(2):
            @pl.when(j < nmine)
            def _():
                s = wid + j * nw
                pltpu.async_copy(tt_hbm.at[:, pl.ds(s * _LANE, _LANE)], src_v.at[j], isem.at[j])

        def do_slab(k, carry):
            s = wid + k * nw
            b = lax.rem(k, 2)
            bvec = jnp.zeros((16,), jnp.int32) + b
            pltpu.make_async_copy(tt_hbm.at[:, pl.ds(0, _LANE)], src_v.at[b], isem.at[b]).wait()

            @pl.when(k >= 2)
            def _():
                pltpu.make_async_copy(comp_v.at[b], out_hbm.at[pl.ds(0, prow)], osem.at[b]).wait()

            def do_col(c, carry2):
                csplat = jnp.zeros((16,), jnp.int32) + c
                for t in range(8):
                    v = plsc.load_gather(src_v, [bvec, csplat, rvecs[t]])
                    plsc.store_scatter(comp_v, [bvec, r2s[t], rp64s[t] + c], v)
                return carry2

            lax.fori_loop(0, hidden, do_col, 0)
            pltpu.async_copy(comp_v.at[b], out_hbm.at[pl.ds(s * prow, prow)], osem.at[b])

            @pl.when(k + 2 < nmine)
            def _():
                s2 = wid + (k + 2) * nw
                pltpu.async_copy(tt_hbm.at[:, pl.ds(s2 * _LANE, _LANE)], src_v.at[b], isem.at[b])

            return carry

        lax.fori_loop(0, nmine, do_slab, 0)

        @pl.when(nmine >= 1)
        def _():
            pltpu.make_async_copy(comp_v.at[0], out_hbm.at[pl.ds(0, prow)], osem.at[lax.rem(nmine + 1, 2)]).wait()

        @pl.when(nmine >= 2)
        def _():
            pltpu.make_async_copy(comp_v.at[0], out_hbm.at[pl.ds(0, prow)], osem.at[lax.rem(nmine, 2)]).wait()

        if tail:
            @pl.when(wid == 0)
            def _():
                pltpu.sync_copy(tail_hbm, tail_v)
                pltpu.sync_copy(tail_v, out_hbm.at[pl.ds(nslab * 64, tail // 2)])

    return transpose_k


@functools.lru_cache(maxsize=None)
def _build_gather(total, hidden, vocab):
    info = plsc.get_sparse_core_info()
    nc, ns = info.num_cores, info.num_subcores
    nw = nc * ns
    per_w = total // nw
    nchunk = per_w // _CHUNK
    assert per_w * nw == total and nchunk * _CHUNK == per_w

    mesh = plsc.VectorSubcoreMesh(core_axis_name="c", subcore_axis_name="s")

    @functools.partial(
        pl.kernel,
        mesh=mesh,
        out_type=jax.ShapeDtypeStruct((total // _CHUNK, _CHUNK, _PADW), jnp.float32),
        compiler_params=pltpu.CompilerParams(use_tc_tiling_on_sc=True),
        scratch_types=[
            pltpu.VMEM((_pad8(nchunk), _CHUNK), jnp.int32),
            pltpu.VMEM((_NBUF, _CHUNK, _PADW), jnp.float32),
            pltpu.SemaphoreType.DMA((_NBUF,)),
        ],
    )
    def gather_k(table_hbm, idx_hbm, out_hbm, idx_v, rows_v, gsem):
        wid = lax.axis_index("s") * nc + lax.axis_index("c")
        cbase = wid * nchunk
        pltpu.sync_copy(idx_hbm.at[wid], idx_v)

        for j in range(_NBUF):
            pltpu.async_copy(table_hbm.at[idx_v.at[j]], rows_v.at[j], gsem.at[j])

        def body(c, carry):
            p = lax.rem(c, _NBUF)
            pltpu.make_async_copy(table_hbm.at[idx_v.at[p]], rows_v.at[p], gsem.at[p]).wait()
            pltpu.sync_copy(rows_v.at[p], out_hbm.at[cbase + c])
            pltpu.async_copy(table_hbm.at[idx_v.at[c + _NBUF]], rows_v.at[p], gsem.at[p])
            return carry

        lax.fori_loop(0, nchunk - _NBUF, body, 0)

        for j in range(nchunk - _NBUF, nchunk):
            p = j % _NBUF
            pltpu.make_async_copy(table_hbm.at[idx_v.at[p]], rows_v.at[p], gsem.at[p]).wait()
            pltpu.sync_copy(rows_v.at[p], out_hbm.at[cbase + j])

    return gather_k


def kernel(inputs, embedding):
    b, h = inputs.shape
    total = b * h
    vocab, hidden = embedding.shape
    nw = 32
    nchunk = total // nw // _CHUNK
    nslab = vocab // _LANE
    tail = vocab - nslab * _LANE

    tt = embedding.T
    tail_op = embedding[nslab * _LANE:].reshape(tail // 2 if tail else 2, _LANE)
    table2 = _build_transpose(hidden, vocab)(tt, tail_op)

    idx = inputs.astype(jnp.int32).reshape(total)
    half3 = (idx >> 1).reshape(nw, nchunk, _CHUNK)
    half3 = jnp.pad(half3, ((0, 0), (0, _pad8(nchunk) - nchunk), (0, 0)))
    pairs = _build_gather(total, hidden, vocab)(table2, half3)
    parity = (idx & 1).reshape(total // _CHUNK, _CHUNK, 1)
    out = jnp.where(parity == 1, pairs[:, :, hidden:], pairs[:, :, :hidden])
    return out.reshape(b, h, hidden)


# speedup vs baseline: 2.4728x; 2.0125x over previous
import functools

import jax
import jax.numpy as jnp
from jax import lax
from jax.experimental import pallas as pl
from jax.experimental.pallas import tpu as pltpu
from jax.experimental.pallas import tpu_sc as plsc

_CHUNK = 128
_NBUF = 4
_PADW = 128


@functools.lru_cache(maxsize=None)
def _build(total, hidden, vocab):
    info = plsc.get_sparse_core_info()
    nc, ns = info.num_cores, info.num_subcores
    nw = nc * ns
    per_w = total // nw
    nchunk = per_w // _CHUNK
    assert per_w * nw == total and nchunk * _CHUNK == per_w

    mesh = plsc.VectorSubcoreMesh(core_axis_name="c", subcore_axis_name="s")

    @functools.partial(
        pl.kernel,
        mesh=mesh,
        out_type=jax.ShapeDtypeStruct((total // _CHUNK, _CHUNK, _PADW), jnp.float32),
        compiler_params=pltpu.CompilerParams(use_tc_tiling_on_sc=False),
        scratch_types=[
            pltpu.VMEM((nchunk, _CHUNK), jnp.int32),
            pltpu.VMEM((_NBUF, _CHUNK, _PADW), jnp.float32),
            pltpu.SemaphoreType.DMA((_NBUF,)),
        ],
    )
    def gather_k(table_hbm, idx_hbm, out_hbm, idx_v, rows_v, gsem):
        wid = lax.axis_index("s") * nc + lax.axis_index("c")
        cbase = wid * nchunk
        pltpu.sync_copy(idx_hbm.at[pl.ds(cbase, nchunk)], idx_v)

        for j in range(_NBUF):
            pltpu.async_copy(table_hbm.at[idx_v.at[j]], rows_v.at[j], gsem.at[j])

        def body(c, carry):
            p = lax.rem(c, _NBUF)
            pltpu.make_async_copy(table_hbm.at[idx_v.at[p]], rows_v.at[p], gsem.at[p]).wait()
            pltpu.sync_copy(rows_v.at[p], out_hbm.at[cbase + c])
            pltpu.async_copy(table_hbm.at[idx_v.at[c + _NBUF]], rows_v.at[p], gsem.at[p])
            return carry

        lax.fori_loop(0, nchunk - _NBUF, body, 0)

        for j in range(nchunk - _NBUF, nchunk):
            p = j % _NBUF
            pltpu.make_async_copy(table_hbm.at[idx_v.at[p]], rows_v.at[p], gsem.at[p]).wait()
            pltpu.sync_copy(rows_v.at[p], out_hbm.at[cbase + j])

    return gather_k


def kernel(inputs, embedding):
    b, h = inputs.shape
    total = b * h
    vocab, hidden = embedding.shape
    idx2 = inputs.astype(jnp.int32).reshape(total // _CHUNK, _CHUNK)
    table = jnp.pad(embedding, ((0, 0), (0, _PADW - hidden)))
    out = _build(total, hidden, vocab)(table, idx2)
    return out[:, :, :hidden].reshape(b, h, hidden)
